# baseline (speedup 1.0000x reference)
import functools

import jax
import jax.numpy as jnp
from jax import lax
from jax.experimental import pallas as pl
from jax.experimental.pallas import tpu as pltpu
from jax.experimental.pallas import tpu_sc as plsc

B = 4096
L = 50
HID = 128
NW = 32
ROWS_W = B // NW
NBUF = 5
AHEAD = 4


def _body(eeg_tab, ecg_tab, eeg_idx, ecg_idx, eeg_out, ecg_out,
          idx_v, *ring):
    bufs = ring[:NBUF]
    gsem = ring[NBUF:2 * NBUF]
    ssem = ring[2 * NBUF:2 * NBUF + NBUF]
    isem = ring[3 * NBUF:]
    wid = lax.axis_index("c") * 16 + lax.axis_index("s")
    row_base = wid * ROWS_W

    idx_cps = [pltpu.make_async_copy(
                   ihbm.at[:, pl.ds(row_base, ROWS_W)], idx_v.at[m], isem[m])
               for m, ihbm in ((0, eeg_idx), (1, ecg_idx))]
    for cp in idx_cps:
        cp.start()

    for m, (tab, out_hbm) in enumerate((
        (eeg_tab, eeg_out),
        (ecg_tab, ecg_out),
    )):
        idx_cps[m].wait()

        def gather(l, b):
            return pltpu.make_async_copy(
                tab.at[idx_v.at[m, l]], bufs[b], gsem[b])

        def store(l, b):
            return pltpu.make_async_copy(
                bufs[b], out_hbm.at[l, pl.ds(row_base, ROWS_W)], ssem[b])

        for l in range(AHEAD):
            gather(l, l).start()

        def turn(jj, _):
            for b in range(NBUF):
                lj = jj * NBUF + b
                @pl.when(lj >= NBUF - AHEAD)
                def _drain():
                    store(lj - (NBUF - AHEAD), (b + AHEAD) % NBUF).wait()

                @pl.when(lj + AHEAD < L)
                def _fire():
                    gather(lj + AHEAD, (b + AHEAD) % NBUF).start()

                gather(lj, b).wait()
                store(lj, b).start()
            return _

        lax.fori_loop(0, L // NBUF, turn, None)

        for l in range(L - (NBUF - AHEAD), L):
            store(l, l % NBUF).wait()


@functools.partial(jax.jit, static_argnums=())
def kernel(eeg_input_ids, ecg_input_ids, eeg_table, ecg_table):
    eeg_idx = eeg_input_ids.astype(jnp.int32).T
    ecg_idx = ecg_input_ids.astype(jnp.int32).T

    mesh = plsc.VectorSubcoreMesh(core_axis_name="c", subcore_axis_name="s")
    run = pl.kernel(
        _body,
        mesh=mesh,
        out_type=[
            jax.ShapeDtypeStruct((L, B, HID), jnp.float32),
            jax.ShapeDtypeStruct((L, B, HID), jnp.float32),
        ],
        scratch_types=(
            [pltpu.VMEM((2, L, ROWS_W), jnp.int32)]
            + [pltpu.VMEM((ROWS_W, HID), jnp.float32)] * NBUF
            + [pltpu.SemaphoreType.DMA] * (3 * NBUF)
        ),
    )
    eeg_t, ecg_t = run(eeg_table, ecg_table, eeg_idx, ecg_idx)
    return (eeg_t.transpose(1, 0, 2), ecg_t.transpose(1, 0, 2))


# speedup vs baseline: 1.0107x; 1.0107x over previous
import functools

import jax
import jax.numpy as jnp
from jax import lax
from jax.experimental import pallas as pl
from jax.experimental.pallas import tpu as pltpu
from jax.experimental.pallas import tpu_sc as plsc

B = 4096
L = 50
HID = 128
NW = 32
ROWS_W = B // NW
NBUF = 5
AHEAD = 3


def _body(eeg_tab, ecg_tab, eeg_idx, ecg_idx, eeg_out, ecg_out,
          idx_v, *ring):
    bufs = ring[:NBUF]
    gsem = ring[NBUF:2 * NBUF]
    ssem = ring[2 * NBUF:]
    wid = lax.axis_index("c") * 16 + lax.axis_index("s")
    row_base = wid * ROWS_W

    for (tab, idx_hbm, out_hbm) in (
        (eeg_tab, eeg_idx, eeg_out),
        (ecg_tab, ecg_idx, ecg_out),
    ):
        pltpu.sync_copy(idx_hbm.at[:, pl.ds(row_base, ROWS_W)], idx_v)

        def gather(l, b):
            return pltpu.make_async_copy(
                tab.at[idx_v.at[l]], bufs[b], gsem[b])

        def store(l, b):
            return pltpu.make_async_copy(
                bufs[b], out_hbm.at[l, pl.ds(row_base, ROWS_W)], ssem[b])

        for l in range(AHEAD):
            gather(l, l).start()

        def turn(jj, _):
            for b in range(NBUF):
                lj = jj * NBUF + b
                @pl.when(lj >= NBUF - AHEAD)
                def _drain():
                    store(lj - (NBUF - AHEAD), (b + AHEAD) % NBUF).wait()

                @pl.when(lj + AHEAD < L)
                def _fire():
                    gather(lj + AHEAD, (b + AHEAD) % NBUF).start()

                gather(lj, b).wait()
                store(lj, b).start()
            return _

        lax.fori_loop(0, L // NBUF, turn, None)

        for l in range(L - (NBUF - AHEAD), L):
            store(l, l % NBUF).wait()


@functools.partial(jax.jit, static_argnums=())
def kernel(eeg_input_ids, ecg_input_ids, eeg_table, ecg_table):
    eeg_idx = eeg_input_ids.astype(jnp.int32).T
    ecg_idx = ecg_input_ids.astype(jnp.int32).T

    mesh = plsc.VectorSubcoreMesh(core_axis_name="c", subcore_axis_name="s")
    run = pl.kernel(
        _body,
        mesh=mesh,
        out_type=[
            jax.ShapeDtypeStruct((L, B, HID), jnp.float32),
            jax.ShapeDtypeStruct((L, B, HID), jnp.float32),
        ],
        scratch_types=(
            [pltpu.VMEM((L, ROWS_W), jnp.int32)]
            + [pltpu.VMEM((ROWS_W, HID), jnp.float32)] * NBUF
            + [pltpu.SemaphoreType.DMA] * (2 * NBUF)
        ),
    )
    eeg_t, ecg_t = run(eeg_table, ecg_table, eeg_idx, ecg_idx)
    return (eeg_t.transpose(1, 0, 2), ecg_t.transpose(1, 0, 2))
